# baseline (speedup 1.0000x reference)
import functools

import jax
import jax.numpy as jnp
from jax import lax
from jax.experimental import pallas as pl
from jax.experimental.pallas import tpu as pltpu
from jax.experimental.pallas import tpu_sc as plsc

N = 10000
D = 128
E = 320000

N_PAD = 10240
ROWS_SUB = N_PAD // 16
NW = 32
K = 128
NCH = 80
E_W = NCH * K
E_PAD = NW * E_W

_MESH = plsc.VectorSubcoreMesh(core_axis_name="c", subcore_axis_name="s")
_SC_PARAMS = pltpu.CompilerParams(needs_layout_passes=False)


@functools.partial(
    pl.kernel,
    out_type=jax.ShapeDtypeStruct((2, N_PAD), jnp.float32),
    mesh=_MESH,
    compiler_params=_SC_PARAMS,
    scratch_types=[
        pltpu.VMEM((NCH, K), jnp.int32),
        pltpu.VMEM((K,), jnp.float32),
        pltpu.VMEM((ROWS_SUB,), jnp.float32),
        pltpu.VMEM_SHARED((N_PAD,), jnp.float32),
    ],
)
def _deg_call(dsts_hbm, out_hbm, didx_v, ones_v, zeros_v, deg_sh):
    cid = lax.axis_index("c")
    sid = lax.axis_index("s")
    wid = cid * 16 + sid
    for i in range(K // 16):
        ones_v[pl.ds(i * 16, 16)] = jnp.full((16,), 1.0, jnp.float32)
    for i in range(ROWS_SUB // 16):
        zeros_v[pl.ds(i * 16, 16)] = jnp.zeros((16,), jnp.float32)
    pltpu.sync_copy(zeros_v, deg_sh.at[pl.ds(sid * ROWS_SUB, ROWS_SUB)])
    pltpu.sync_copy(dsts_hbm.at[wid], didx_v)
    plsc.subcore_barrier()

    def body(j, carry):
        pltpu.sync_copy(ones_v, deg_sh.at[didx_v.at[j]], add=True)
        return carry

    lax.fori_loop(0, NCH, body, 0)
    plsc.subcore_barrier()
    pltpu.sync_copy(deg_sh.at[pl.ds(sid * ROWS_SUB, ROWS_SUB)],
                    out_hbm.at[cid, pl.ds(sid * ROWS_SUB, ROWS_SUB)])


@functools.partial(
    pl.kernel,
    out_type=jax.ShapeDtypeStruct((2, N_PAD, D), jnp.float32),
    mesh=_MESH,
    compiler_params=_SC_PARAMS,
    scratch_types=[
        pltpu.VMEM((4, K), jnp.int32),
        pltpu.VMEM((4, K), jnp.int32),
        pltpu.VMEM((K, D), jnp.float32),
        pltpu.VMEM((K, D), jnp.float32),
        pltpu.SemaphoreType.DMA,
        pltpu.SemaphoreType.DMA,
        pltpu.SemaphoreType.DMA,
        pltpu.SemaphoreType.DMA,
        pltpu.SemaphoreType.DMA,
        pltpu.SemaphoreType.DMA,
        pltpu.SemaphoreType.DMA,
        pltpu.SemaphoreType.DMA,
        pltpu.VMEM_SHARED((N_PAD, D), jnp.float32),
    ],
)
def _scatter_call(z_hbm, srcs_hbm, dsts_hbm, out_hbm,
                  sidx, didx, rows0, rows1,
                  i0, i1, i2, i3, g0, g1, p0, p1, acc_sh):
    cid = lax.axis_index("c")
    sid = lax.axis_index("s")
    wid = cid * 16 + sid
    rows = (rows0, rows1)
    isem = (i0, i1, i2, i3)
    gsem = (g0, g1)
    ssem = (p0, p1)

    pltpu.sync_copy(z_hbm.at[pl.ds(sid * ROWS_SUB, ROWS_SUB)],
                    acc_sh.at[pl.ds(sid * ROWS_SUB, ROWS_SUB)])
    plsc.subcore_barrier()

    def issue_idx(j, q):
        pltpu.async_copy(srcs_hbm.at[wid, j], sidx.at[q], isem[q])
        pltpu.async_copy(dsts_hbm.at[wid, j], didx.at[q], isem[q])

    def wait_idx(j, q):
        pltpu.make_async_copy(srcs_hbm.at[wid, j], sidx.at[q], isem[q]).wait()
        pltpu.make_async_copy(dsts_hbm.at[wid, j], didx.at[q], isem[q]).wait()

    def issue_gather(q, b):
        pltpu.async_copy(z_hbm.at[sidx.at[q]], rows[b], gsem[b])

    def wait_gather(q, b):
        pltpu.make_async_copy(z_hbm.at[sidx.at[q]], rows[b], gsem[b]).wait()

    def issue_scatter(q, b):
        pltpu.async_copy(rows[b], acc_sh.at[didx.at[q]], ssem[b], add=True)

    def wait_scatter(q, b):
        pltpu.make_async_copy(rows[b], acc_sh.at[didx.at[q]], ssem[b]).wait()

    issue_idx(0, 0)
    issue_idx(1, 1)
    issue_idx(2, 2)
    wait_idx(0, 0)
    issue_gather(0, 0)

    def body(jj, carry):
        for u in range(4):
            j = 4 * jj + u
            b = u % 2
            b1 = 1 - b
            q = u
            q1 = (u + 1) % 4
            q3 = (u + 3) % 4
            wait_gather(q, b)

            if u == 0:
                @pl.when(jj > 0)
                def _():
                    wait_scatter(q3, b1)
            else:
                wait_scatter(q3, b1)

            issue_scatter(q, b)

            @pl.when(j < NCH - 1)
            def _():
                wait_idx(j + 1, q1)
                issue_gather(q1, b1)

            @pl.when(j < NCH - 3)
            def _():
                issue_idx(j + 3, q3)

        return carry

    lax.fori_loop(0, NCH // 4, body, 0)
    wait_scatter(3, 1)
    plsc.subcore_barrier()
    pltpu.sync_copy(acc_sh.at[pl.ds(sid * ROWS_SUB, ROWS_SUB)],
                    out_hbm.at[cid, pl.ds(sid * ROWS_SUB, ROWS_SUB)])


@functools.partial(
    pl.kernel,
    out_type=jax.ShapeDtypeStruct((NW, NCH, K), jnp.float32),
    mesh=_MESH,
    compiler_params=_SC_PARAMS,
    scratch_types=[
        pltpu.VMEM((NCH, K), jnp.int32),
        pltpu.VMEM((NCH, K), jnp.int32),
        pltpu.VMEM((K, D), jnp.float32),
        pltpu.VMEM((K, D), jnp.float32),
        pltpu.VMEM((K, D), jnp.float32),
        pltpu.VMEM((K, D), jnp.float32),
        pltpu.VMEM((NCH, K), jnp.float32),
        pltpu.SemaphoreType.DMA,
        pltpu.SemaphoreType.DMA,
    ],
)
def _logits_call(h_hbm, aidx_hbm, bidx_hbm, out_hbm,
                 aidx_v, bidx_v, ra0, ra1, rb0, rb1, out_v, s0, s1):
    cid = lax.axis_index("c")
    sid = lax.axis_index("s")
    wid = cid * 16 + sid
    ra = (ra0, ra1)
    rb = (rb0, rb1)
    sem = (s0, s1)

    pltpu.sync_copy(aidx_hbm.at[wid], aidx_v)
    pltpu.sync_copy(bidx_hbm.at[wid], bidx_v)

    def issue(j, b):
        pltpu.async_copy(h_hbm.at[aidx_v.at[j]], ra[b], sem[b])
        pltpu.async_copy(h_hbm.at[bidx_v.at[j]], rb[b], sem[b])

    def wait(j, b):
        pltpu.make_async_copy(h_hbm.at[aidx_v.at[j]], ra[b], sem[b]).wait()
        pltpu.make_async_copy(h_hbm.at[bidx_v.at[j]], rb[b], sem[b]).wait()

    issue(0, 0)
    lane = lax.broadcasted_iota(jnp.int32, (16,), 0)

    def compute(j, b):
        def group(g, c):
            def edot(t, accv):
                e = g * 16 + t
                acc = ra[b][e, pl.ds(0, 16)] * rb[b][e, pl.ds(0, 16)]
                for k in range(1, D // 16):
                    acc = acc + ra[b][e, pl.ds(k * 16, 16)] * rb[b][e, pl.ds(k * 16, 16)]
                return jnp.where(lane == t, jnp.sum(acc), accv)

            accv = lax.fori_loop(0, 16, edot, jnp.zeros((16,), jnp.float32),
                                 unroll=4)
            out_v[j, pl.ds(g * 16, 16)] = accv
            return c

        lax.fori_loop(0, K // 16, group, 0, unroll=2)

    def body(jj, carry):
        j = 2 * jj
        wait(j, 0)
        issue(j + 1, 1)
        compute(j, 0)
        wait(j + 1, 1)

        @pl.when(jj < NCH // 2 - 1)
        def _():
            issue(j + 2, 0)

        compute(j + 1, 1)
        return carry

    lax.fori_loop(0, NCH // 2, body, 0)
    pltpu.sync_copy(out_v, out_hbm.at[wid])


_BLK = 1280
_GRID = N_PAD // _BLK

_row_spec = pl.BlockSpec((_BLK, D), lambda i: (i, 0))
_vec_spec = pl.BlockSpec((_BLK, 1), lambda i: (i, 0))
_full_mat = pl.BlockSpec((D, D), lambda i: (0, 0))
_full_vec = pl.BlockSpec((D,), lambda i: (0,))


def _dinv(p0, p1):
    return lax.rsqrt(1.0 + p0 + p1)


def _tc_a_body(x_ref, wl_ref, bl_ref, w1_ref, p0_ref, p1_ref, z_ref):
    t = jnp.dot(x_ref[...], wl_ref[...], preferred_element_type=jnp.float32)
    t = t + bl_ref[...][None, :]
    y = jnp.dot(t, w1_ref[...], preferred_element_type=jnp.float32)
    z_ref[...] = y * _dinv(p0_ref[...], p1_ref[...])


_tc_a = pl.pallas_call(
    _tc_a_body,
    grid=(_GRID,),
    in_specs=[_row_spec, _full_mat, _full_vec, _full_mat, _vec_spec, _vec_spec],
    out_specs=_row_spec,
    out_shape=jax.ShapeDtypeStruct((N_PAD, D), jnp.float32),
)


def _tc_b_body(sa_ref, sb_ref, z1_ref, p0_ref, p1_ref, b1_ref, w2_ref, z2_ref):
    dinv = _dinv(p0_ref[...], p1_ref[...])
    s = sa_ref[...] + sb_ref[...] - z1_ref[...]
    x1 = jnp.maximum(dinv * s + b1_ref[...][None, :], 0.0)
    y2 = jnp.dot(x1, w2_ref[...], preferred_element_type=jnp.float32)
    z2_ref[...] = y2 * dinv


_tc_b = pl.pallas_call(
    _tc_b_body,
    grid=(_GRID,),
    in_specs=[_row_spec, _row_spec, _row_spec, _vec_spec, _vec_spec,
              _full_vec, _full_mat],
    out_specs=_row_spec,
    out_shape=jax.ShapeDtypeStruct((N_PAD, D), jnp.float32),
)


def _tc_c_body(sa_ref, sb_ref, z2_ref, p0_ref, p1_ref, b2_ref, h_ref):
    dinv = _dinv(p0_ref[...], p1_ref[...])
    s = sa_ref[...] + sb_ref[...] - z2_ref[...]
    h_ref[...] = dinv * s + b2_ref[...][None, :]


_tc_c = pl.pallas_call(
    _tc_c_body,
    grid=(_GRID,),
    in_specs=[_row_spec, _row_spec, _row_spec, _vec_spec, _vec_spec, _full_vec],
    out_specs=_row_spec,
    out_shape=jax.ShapeDtypeStruct((N_PAD, D), jnp.float32),
)


def _pack_edges(v, pad_vals):
    return jnp.concatenate([v, pad_vals]).reshape(NW, NCH, K)


def kernel(x_input, edge_index_input, pos_edge_index_input,
           W_lin, b_lin, W1, b1, W2, b2):
    x_pad = jnp.zeros((N_PAD, D), jnp.float32).at[:N].set(x_input)
    pos = pos_edge_index_input.astype(jnp.int32)
    ei = edge_index_input.astype(jnp.int32)
    pad_vals = N + (jnp.arange(E_PAD - E, dtype=jnp.int32) % 16)
    srcs = _pack_edges(pos[0], pad_vals)
    dsts = _pack_edges(pos[1], pad_vals)
    aidx = _pack_edges(ei[0], pad_vals)
    bidx = _pack_edges(ei[1], pad_vals)

    degp = _deg_call(dsts)
    p0, p1 = degp[0][:, None], degp[1][:, None]

    z1 = _tc_a(x_pad, W_lin, b_lin, W1, p0, p1)
    s1 = _scatter_call(z1, srcs, dsts)
    z2 = _tc_b(s1[0], s1[1], z1, p0, p1, b1, W2)
    s2 = _scatter_call(z2, srcs, dsts)
    h2 = _tc_c(s2[0], s2[1], z2, p0, p1, b2)

    logits = _logits_call(h2, aidx, bidx)
    return logits.reshape(-1)[:E]


# speedup vs baseline: 1.0006x; 1.0006x over previous
import functools

import jax
import jax.numpy as jnp
from jax import lax
from jax.experimental import pallas as pl
from jax.experimental.pallas import tpu as pltpu
from jax.experimental.pallas import tpu_sc as plsc

N = 10000
D = 128
E = 320000

N_PAD = 10240
ROWS_SUB = N_PAD // 16
NW = 32
K = 128
NCH = 80
E_W = NCH * K
E_PAD = NW * E_W

_MESH = plsc.VectorSubcoreMesh(core_axis_name="c", subcore_axis_name="s")
_SC_PARAMS = pltpu.CompilerParams(needs_layout_passes=False)


@functools.partial(
    pl.kernel,
    out_type=jax.ShapeDtypeStruct((2, N_PAD), jnp.float32),
    mesh=_MESH,
    compiler_params=_SC_PARAMS,
    scratch_types=[
        pltpu.VMEM((NCH, K), jnp.int32),
        pltpu.VMEM((K,), jnp.float32),
        pltpu.VMEM((ROWS_SUB,), jnp.float32),
        pltpu.VMEM_SHARED((N_PAD,), jnp.float32),
    ],
)
def _deg_call(dsts_hbm, out_hbm, didx_v, ones_v, zeros_v, deg_sh):
    cid = lax.axis_index("c")
    sid = lax.axis_index("s")
    wid = cid * 16 + sid
    for i in range(K // 16):
        ones_v[pl.ds(i * 16, 16)] = jnp.full((16,), 1.0, jnp.float32)
    for i in range(ROWS_SUB // 16):
        zeros_v[pl.ds(i * 16, 16)] = jnp.zeros((16,), jnp.float32)
    pltpu.sync_copy(zeros_v, deg_sh.at[pl.ds(sid * ROWS_SUB, ROWS_SUB)])
    pltpu.sync_copy(dsts_hbm.at[wid], didx_v)
    plsc.subcore_barrier()

    def body(j, carry):
        pltpu.sync_copy(ones_v, deg_sh.at[didx_v.at[j]], add=True)
        return carry

    lax.fori_loop(0, NCH, body, 0)
    plsc.subcore_barrier()
    pltpu.sync_copy(deg_sh.at[pl.ds(sid * ROWS_SUB, ROWS_SUB)],
                    out_hbm.at[cid, pl.ds(sid * ROWS_SUB, ROWS_SUB)])


@functools.partial(
    pl.kernel,
    out_type=jax.ShapeDtypeStruct((2, N_PAD, D), jnp.float32),
    mesh=_MESH,
    compiler_params=_SC_PARAMS,
    scratch_types=[
        pltpu.VMEM((4, K), jnp.int32),
        pltpu.VMEM((4, K), jnp.int32),
        pltpu.VMEM((K, D), jnp.float32),
        pltpu.VMEM((K, D), jnp.float32),
        pltpu.SemaphoreType.DMA,
        pltpu.SemaphoreType.DMA,
        pltpu.SemaphoreType.DMA,
        pltpu.SemaphoreType.DMA,
        pltpu.SemaphoreType.DMA,
        pltpu.SemaphoreType.DMA,
        pltpu.SemaphoreType.DMA,
        pltpu.SemaphoreType.DMA,
        pltpu.VMEM_SHARED((N_PAD, D), jnp.float32),
    ],
)
def _scatter_call(z_hbm, srcs_hbm, dsts_hbm, out_hbm,
                  sidx, didx, rows0, rows1,
                  i0, i1, i2, i3, g0, g1, p0, p1, acc_sh):
    cid = lax.axis_index("c")
    sid = lax.axis_index("s")
    wid = cid * 16 + sid
    rows = (rows0, rows1)
    isem = (i0, i1, i2, i3)
    gsem = (g0, g1)
    ssem = (p0, p1)

    pltpu.sync_copy(z_hbm.at[pl.ds(sid * ROWS_SUB, ROWS_SUB)],
                    acc_sh.at[pl.ds(sid * ROWS_SUB, ROWS_SUB)])
    plsc.subcore_barrier()

    def issue_idx(j, q):
        pltpu.async_copy(srcs_hbm.at[wid, j], sidx.at[q], isem[q])
        pltpu.async_copy(dsts_hbm.at[wid, j], didx.at[q], isem[q])

    def wait_idx(j, q):
        pltpu.make_async_copy(srcs_hbm.at[wid, j], sidx.at[q], isem[q]).wait()
        pltpu.make_async_copy(dsts_hbm.at[wid, j], didx.at[q], isem[q]).wait()

    def issue_gather(q, b):
        pltpu.async_copy(z_hbm.at[sidx.at[q]], rows[b], gsem[b])

    def wait_gather(q, b):
        pltpu.make_async_copy(z_hbm.at[sidx.at[q]], rows[b], gsem[b]).wait()

    def issue_scatter(q, b):
        pltpu.async_copy(rows[b], acc_sh.at[didx.at[q]], ssem[b], add=True)

    def wait_scatter(q, b):
        pltpu.make_async_copy(rows[b], acc_sh.at[didx.at[q]], ssem[b]).wait()

    issue_idx(0, 0)
    issue_idx(1, 1)
    issue_idx(2, 2)
    wait_idx(0, 0)
    issue_gather(0, 0)

    def body(jj, carry):
        for u in range(4):
            j = 4 * jj + u
            b = u % 2
            b1 = 1 - b
            q = u
            q1 = (u + 1) % 4
            q3 = (u + 3) % 4
            wait_gather(q, b)

            if u == 0:
                @pl.when(jj > 0)
                def _():
                    wait_scatter(q3, b1)
            else:
                wait_scatter(q3, b1)

            issue_scatter(q, b)

            @pl.when(j < NCH - 1)
            def _():
                wait_idx(j + 1, q1)
                issue_gather(q1, b1)

            @pl.when(j < NCH - 3)
            def _():
                issue_idx(j + 3, q3)

        return carry

    lax.fori_loop(0, NCH // 4, body, 0)
    wait_scatter(3, 1)
    plsc.subcore_barrier()
    pltpu.sync_copy(acc_sh.at[pl.ds(sid * ROWS_SUB, ROWS_SUB)],
                    out_hbm.at[cid, pl.ds(sid * ROWS_SUB, ROWS_SUB)])


LK = 64
LNCH = E_W // LK


@functools.partial(
    pl.kernel,
    out_type=jax.ShapeDtypeStruct((NW, LNCH, LK), jnp.float32),
    mesh=_MESH,
    compiler_params=_SC_PARAMS,
    scratch_types=[
        pltpu.VMEM((2, LK), jnp.int32),
        pltpu.VMEM((2, LK), jnp.int32),
        pltpu.VMEM((2, LK, D), jnp.float32),
        pltpu.VMEM((2, LK, D), jnp.float32),
        pltpu.VMEM((2, LK), jnp.float32),
        pltpu.VMEM_SHARED((N_PAD, D), jnp.float32),
        pltpu.SemaphoreType.DMA,
        pltpu.SemaphoreType.DMA,
        pltpu.SemaphoreType.DMA,
        pltpu.SemaphoreType.DMA,
        pltpu.SemaphoreType.DMA,
        pltpu.SemaphoreType.DMA,
    ],
)
def _logits_call(h_hbm, aidx_hbm, bidx_hbm, out_hbm,
                 aidx_v, bidx_v, ra, rb, oc, h_sh,
                 s0, s1, si0, si1, so0, so1):
    cid = lax.axis_index("c")
    sid = lax.axis_index("s")
    wid = cid * 16 + sid
    sem = (s0, s1)
    isem = (si0, si1)
    osem = (so0, so1)

    pltpu.sync_copy(h_hbm.at[pl.ds(sid * ROWS_SUB, ROWS_SUB)],
                    h_sh.at[pl.ds(sid * ROWS_SUB, ROWS_SUB)])
    plsc.subcore_barrier()

    def issue_idx(j, b):
        pltpu.async_copy(aidx_hbm.at[wid, j], aidx_v.at[b], isem[b])
        pltpu.async_copy(bidx_hbm.at[wid, j], bidx_v.at[b], isem[b])

    def wait_idx(j, b):
        pltpu.make_async_copy(aidx_hbm.at[wid, j], aidx_v.at[b], isem[b]).wait()
        pltpu.make_async_copy(bidx_hbm.at[wid, j], bidx_v.at[b], isem[b]).wait()

    def issue(b):
        pltpu.async_copy(h_sh.at[aidx_v.at[b]], ra.at[b], sem[b])
        pltpu.async_copy(h_sh.at[bidx_v.at[b]], rb.at[b], sem[b])

    def wait(b):
        pltpu.make_async_copy(h_sh.at[aidx_v.at[b]], ra.at[b], sem[b]).wait()
        pltpu.make_async_copy(h_sh.at[bidx_v.at[b]], rb.at[b], sem[b]).wait()

    def issue_out(j, b):
        pltpu.async_copy(oc.at[b], out_hbm.at[wid, j], osem[b])

    def wait_out(b):
        pltpu.make_async_copy(oc.at[b], out_hbm.at[wid, 0], osem[b]).wait()

    lane = lax.broadcasted_iota(jnp.int32, (16,), 0)

    issue_idx(0, 0)
    wait_idx(0, 0)
    issue(0)
    issue_idx(1, 1)

    def compute(j, b):
        def group(g, c):
            def edot(t, accv):
                e = g * 16 + t
                acc = ra[b, e, pl.ds(0, 16)] * rb[b, e, pl.ds(0, 16)]
                for k in range(1, D // 16):
                    acc = acc + ra[b, e, pl.ds(k * 16, 16)] * rb[b, e, pl.ds(k * 16, 16)]
                return jnp.where(lane == t, jnp.sum(acc), accv)

            accv = lax.fori_loop(0, 16, edot, jnp.zeros((16,), jnp.float32),
                                 unroll=4)
            oc[b, pl.ds(g * 16, 16)] = accv
            return c

        lax.fori_loop(0, LK // 16, group, 0, unroll=2)

    def body(jj, carry):
        for u in range(2):
            j = 2 * jj + u
            b = u
            b1 = 1 - u
            wait(b)

            @pl.when(j < LNCH - 1)
            def _():
                wait_idx(j + 1, b1)
                issue(b1)

            @pl.when(j >= 2)
            def _():
                wait_out(b)

            compute(j, b)
            issue_out(j, b)

            @pl.when(j < LNCH - 2)
            def _():
                issue_idx(j + 2, b)

        return carry

    lax.fori_loop(0, LNCH // 2, body, 0)
    wait_out(0)
    wait_out(1)


_BLK = 1280
_GRID = N_PAD // _BLK

_row_spec = pl.BlockSpec((_BLK, D), lambda i: (i, 0))
_vec_spec = pl.BlockSpec((_BLK, 1), lambda i: (i, 0))
_full_mat = pl.BlockSpec((D, D), lambda i: (0, 0))
_full_vec = pl.BlockSpec((D,), lambda i: (0,))


def _dinv(p0, p1):
    return lax.rsqrt(1.0 + p0 + p1)


def _tc_a_body(x_ref, wl_ref, bl_ref, w1_ref, p0_ref, p1_ref, z_ref):
    t = jnp.dot(x_ref[...], wl_ref[...], preferred_element_type=jnp.float32)
    t = t + bl_ref[...][None, :]
    y = jnp.dot(t, w1_ref[...], preferred_element_type=jnp.float32)
    z_ref[...] = y * _dinv(p0_ref[...], p1_ref[...])


_tc_a = pl.pallas_call(
    _tc_a_body,
    grid=(_GRID,),
    in_specs=[_row_spec, _full_mat, _full_vec, _full_mat, _vec_spec, _vec_spec],
    out_specs=_row_spec,
    out_shape=jax.ShapeDtypeStruct((N_PAD, D), jnp.float32),
)


def _tc_b_body(sa_ref, sb_ref, z1_ref, p0_ref, p1_ref, b1_ref, w2_ref, z2_ref):
    dinv = _dinv(p0_ref[...], p1_ref[...])
    s = sa_ref[...] + sb_ref[...] - z1_ref[...]
    x1 = jnp.maximum(dinv * s + b1_ref[...][None, :], 0.0)
    y2 = jnp.dot(x1, w2_ref[...], preferred_element_type=jnp.float32)
    z2_ref[...] = y2 * dinv


_tc_b = pl.pallas_call(
    _tc_b_body,
    grid=(_GRID,),
    in_specs=[_row_spec, _row_spec, _row_spec, _vec_spec, _vec_spec,
              _full_vec, _full_mat],
    out_specs=_row_spec,
    out_shape=jax.ShapeDtypeStruct((N_PAD, D), jnp.float32),
)


def _tc_c_body(sa_ref, sb_ref, z2_ref, p0_ref, p1_ref, b2_ref, h_ref):
    dinv = _dinv(p0_ref[...], p1_ref[...])
    s = sa_ref[...] + sb_ref[...] - z2_ref[...]
    h_ref[...] = dinv * s + b2_ref[...][None, :]


_tc_c = pl.pallas_call(
    _tc_c_body,
    grid=(_GRID,),
    in_specs=[_row_spec, _row_spec, _row_spec, _vec_spec, _vec_spec, _full_vec],
    out_specs=_row_spec,
    out_shape=jax.ShapeDtypeStruct((N_PAD, D), jnp.float32),
)


def _pack_edges(v, pad_vals):
    return jnp.concatenate([v, pad_vals]).reshape(NW, NCH, K)


def kernel(x_input, edge_index_input, pos_edge_index_input,
           W_lin, b_lin, W1, b1, W2, b2):
    x_pad = jnp.zeros((N_PAD, D), jnp.float32).at[:N].set(x_input)
    pos = pos_edge_index_input.astype(jnp.int32)
    ei = edge_index_input.astype(jnp.int32)
    pad_vals = N + (jnp.arange(E_PAD - E, dtype=jnp.int32) % 16)
    srcs = _pack_edges(pos[0], pad_vals)
    dsts = _pack_edges(pos[1], pad_vals)
    aidx = _pack_edges(ei[0], pad_vals)
    bidx = _pack_edges(ei[1], pad_vals)

    degp = _deg_call(dsts)
    p0, p1 = degp[0][:, None], degp[1][:, None]

    z1 = _tc_a(x_pad, W_lin, b_lin, W1, p0, p1)
    s1 = _scatter_call(z1, srcs, dsts)
    z2 = _tc_b(s1[0], s1[1], z1, p0, p1, b1, W2)
    s2 = _scatter_call(z2, srcs, dsts)
    h2 = _tc_c(s2[0], s2[1], z2, p0, p1, b2)

    logits = _logits_call(h2, aidx.reshape(NW, LNCH, LK),
                          bidx.reshape(NW, LNCH, LK))
    return logits.reshape(-1)[:E]


# speedup vs baseline: 1.0406x; 1.0399x over previous
import functools

import jax
import jax.numpy as jnp
from jax import lax
from jax.experimental import pallas as pl
from jax.experimental.pallas import tpu as pltpu
from jax.experimental.pallas import tpu_sc as plsc

N = 10000
D = 128
E = 320000

N_PAD = 10240
ROWS_SUB = N_PAD // 16
NW = 32
K = 128
NCH = 80
E_W = NCH * K
E_PAD = NW * E_W

_MESH = plsc.VectorSubcoreMesh(core_axis_name="c", subcore_axis_name="s")
_SC_PARAMS = pltpu.CompilerParams(needs_layout_passes=False)


@functools.partial(
    pl.kernel,
    out_type=jax.ShapeDtypeStruct((2, N_PAD), jnp.float32),
    mesh=_MESH,
    compiler_params=_SC_PARAMS,
    scratch_types=[
        pltpu.VMEM((NCH, K), jnp.int32),
        pltpu.VMEM((K,), jnp.float32),
        pltpu.VMEM((ROWS_SUB,), jnp.float32),
        pltpu.VMEM_SHARED((N_PAD,), jnp.float32),
    ],
)
def _deg_call(dsts_hbm, out_hbm, didx_v, ones_v, zeros_v, deg_sh):
    cid = lax.axis_index("c")
    sid = lax.axis_index("s")
    wid = cid * 16 + sid
    for i in range(K // 16):
        ones_v[pl.ds(i * 16, 16)] = jnp.full((16,), 1.0, jnp.float32)
    for i in range(ROWS_SUB // 16):
        zeros_v[pl.ds(i * 16, 16)] = jnp.zeros((16,), jnp.float32)
    pltpu.sync_copy(zeros_v, deg_sh.at[pl.ds(sid * ROWS_SUB, ROWS_SUB)])
    pltpu.sync_copy(dsts_hbm.at[wid], didx_v)
    plsc.subcore_barrier()

    def body(j, carry):
        pltpu.sync_copy(ones_v, deg_sh.at[didx_v.at[j]], add=True)
        return carry

    lax.fori_loop(0, NCH, body, 0)
    plsc.subcore_barrier()
    pltpu.sync_copy(deg_sh.at[pl.ds(sid * ROWS_SUB, ROWS_SUB)],
                    out_hbm.at[cid, pl.ds(sid * ROWS_SUB, ROWS_SUB)])


@functools.partial(
    pl.kernel,
    out_type=jax.ShapeDtypeStruct((2, N_PAD, D), jnp.float32),
    mesh=_MESH,
    compiler_params=_SC_PARAMS,
    scratch_types=[
        pltpu.VMEM((4, K), jnp.int32),
        pltpu.VMEM((4, K), jnp.int32),
        pltpu.VMEM((K, D), jnp.float32),
        pltpu.VMEM((K, D), jnp.float32),
        pltpu.SemaphoreType.DMA,
        pltpu.SemaphoreType.DMA,
        pltpu.SemaphoreType.DMA,
        pltpu.SemaphoreType.DMA,
        pltpu.SemaphoreType.DMA,
        pltpu.SemaphoreType.DMA,
        pltpu.SemaphoreType.DMA,
        pltpu.SemaphoreType.DMA,
        pltpu.VMEM_SHARED((N_PAD, D), jnp.float32),
    ],
)
def _scatter_call(z_hbm, srcs_hbm, dsts_hbm, out_hbm,
                  sidx, didx, rows0, rows1,
                  i0, i1, i2, i3, g0, g1, p0, p1, acc_sh):
    cid = lax.axis_index("c")
    sid = lax.axis_index("s")
    wid = cid * 16 + sid
    rows = (rows0, rows1)
    isem = (i0, i1, i2, i3)
    gsem = (g0, g1)
    ssem = (p0, p1)

    pltpu.sync_copy(z_hbm.at[pl.ds(sid * ROWS_SUB, ROWS_SUB)],
                    acc_sh.at[pl.ds(sid * ROWS_SUB, ROWS_SUB)])
    plsc.subcore_barrier()

    def issue_idx(j, q):
        pltpu.async_copy(srcs_hbm.at[wid, j], sidx.at[q], isem[q])
        pltpu.async_copy(dsts_hbm.at[wid, j], didx.at[q], isem[q])

    def wait_idx(j, q):
        pltpu.make_async_copy(srcs_hbm.at[wid, j], sidx.at[q], isem[q]).wait()
        pltpu.make_async_copy(dsts_hbm.at[wid, j], didx.at[q], isem[q]).wait()

    def issue_gather(q, b):
        pltpu.async_copy(z_hbm.at[sidx.at[q]], rows[b], gsem[b])

    def wait_gather(q, b):
        pltpu.make_async_copy(z_hbm.at[sidx.at[q]], rows[b], gsem[b]).wait()

    def issue_scatter(q, b):
        pltpu.async_copy(rows[b], acc_sh.at[didx.at[q]], ssem[b], add=True)

    def wait_scatter(q, b):
        pltpu.make_async_copy(rows[b], acc_sh.at[didx.at[q]], ssem[b]).wait()

    issue_idx(0, 0)
    issue_idx(1, 1)
    issue_idx(2, 2)
    wait_idx(0, 0)
    issue_gather(0, 0)

    def body(jj, carry):
        for u in range(4):
            j = 4 * jj + u
            b = u % 2
            b1 = 1 - b
            q = u
            q1 = (u + 1) % 4
            q3 = (u + 3) % 4
            wait_gather(q, b)

            if u == 0:
                @pl.when(jj > 0)
                def _():
                    wait_scatter(q3, b1)
            else:
                wait_scatter(q3, b1)

            issue_scatter(q, b)

            @pl.when(j < NCH - 1)
            def _():
                wait_idx(j + 1, q1)
                issue_gather(q1, b1)

            @pl.when(j < NCH - 3)
            def _():
                issue_idx(j + 3, q3)

        return carry

    lax.fori_loop(0, NCH // 4, body, 0)
    wait_scatter(3, 1)
    plsc.subcore_barrier()
    pltpu.sync_copy(acc_sh.at[pl.ds(sid * ROWS_SUB, ROWS_SUB)],
                    out_hbm.at[cid, pl.ds(sid * ROWS_SUB, ROWS_SUB)])


LK = 64
LNCH = E_W // LK


@functools.partial(
    pl.kernel,
    out_type=jax.ShapeDtypeStruct((NW, LNCH, LK), jnp.float32),
    mesh=_MESH,
    compiler_params=pltpu.CompilerParams(needs_layout_passes=False,
                                         use_tc_tiling_on_sc=False),
    scratch_types=[
        pltpu.VMEM((2, LK), jnp.int32),
        pltpu.VMEM((2, LK), jnp.int32),
        pltpu.VMEM((2, LK, D), jnp.bfloat16),
        pltpu.VMEM((2, LK, D), jnp.bfloat16),
        pltpu.VMEM((2, LK), jnp.float32),
        pltpu.VMEM_SHARED((N_PAD, D), jnp.bfloat16),
        pltpu.SemaphoreType.DMA,
        pltpu.SemaphoreType.DMA,
        pltpu.SemaphoreType.DMA,
        pltpu.SemaphoreType.DMA,
        pltpu.SemaphoreType.DMA,
        pltpu.SemaphoreType.DMA,
    ],
)
def _logits_call(h_hbm, aidx_hbm, bidx_hbm, out_hbm,
                 aidx_v, bidx_v, ra, rb, oc, h_sh,
                 s0, s1, si0, si1, so0, so1):
    cid = lax.axis_index("c")
    sid = lax.axis_index("s")
    wid = cid * 16 + sid
    sem = (s0, s1)
    isem = (si0, si1)
    osem = (so0, so1)

    pltpu.sync_copy(h_hbm.at[pl.ds(sid * ROWS_SUB, ROWS_SUB)],
                    h_sh.at[pl.ds(sid * ROWS_SUB, ROWS_SUB)])
    plsc.subcore_barrier()

    def issue_idx(j, b):
        pltpu.async_copy(aidx_hbm.at[wid, j], aidx_v.at[b], isem[b])
        pltpu.async_copy(bidx_hbm.at[wid, j], bidx_v.at[b], isem[b])

    def wait_idx(j, b):
        pltpu.make_async_copy(aidx_hbm.at[wid, j], aidx_v.at[b], isem[b]).wait()
        pltpu.make_async_copy(bidx_hbm.at[wid, j], bidx_v.at[b], isem[b]).wait()

    def issue(b):
        pltpu.async_copy(h_sh.at[aidx_v.at[b]], ra.at[b], sem[b])
        pltpu.async_copy(h_sh.at[bidx_v.at[b]], rb.at[b], sem[b])

    def wait(b):
        pltpu.make_async_copy(h_sh.at[aidx_v.at[b]], ra.at[b], sem[b]).wait()
        pltpu.make_async_copy(h_sh.at[bidx_v.at[b]], rb.at[b], sem[b]).wait()

    def issue_out(j, b):
        pltpu.async_copy(oc.at[b], out_hbm.at[wid, j], osem[b])

    def wait_out(b):
        pltpu.make_async_copy(oc.at[b], out_hbm.at[wid, 0], osem[b]).wait()

    lane = lax.broadcasted_iota(jnp.int32, (16,), 0)

    issue_idx(0, 0)
    wait_idx(0, 0)
    issue(0)
    issue_idx(1, 1)

    def compute(j, b):
        def group(g, c):
            def edot(t, accv):
                e = g * 16 + t
                acc = jnp.zeros((16,), jnp.float32)
                for k in range(D // 32):
                    va = ra[b, e, pl.ds(k * 32, 32)]
                    vb = rb[b, e, pl.ds(k * 32, 32)]
                    va0, va1 = plsc.unpack(
                        va, format=plsc.PackFormat.INTERLEAVED,
                        preferred_element_type=jnp.float32)
                    vb0, vb1 = plsc.unpack(
                        vb, format=plsc.PackFormat.INTERLEAVED,
                        preferred_element_type=jnp.float32)
                    acc = acc + va0 * vb0 + va1 * vb1
                return jnp.where(lane == t, jnp.sum(acc), accv)

            accv = lax.fori_loop(0, 16, edot, jnp.zeros((16,), jnp.float32),
                                 unroll=4)
            oc[b, pl.ds(g * 16, 16)] = accv
            return c

        lax.fori_loop(0, LK // 16, group, 0, unroll=2)

    def body(jj, carry):
        for u in range(2):
            j = 2 * jj + u
            b = u
            b1 = 1 - u
            wait(b)

            @pl.when(j < LNCH - 1)
            def _():
                wait_idx(j + 1, b1)
                issue(b1)

            @pl.when(j >= 2)
            def _():
                wait_out(b)

            compute(j, b)
            issue_out(j, b)

            @pl.when(j < LNCH - 2)
            def _():
                issue_idx(j + 2, b)

        return carry

    lax.fori_loop(0, LNCH // 2, body, 0)
    wait_out(0)
    wait_out(1)


_BLK = 1280
_GRID = N_PAD // _BLK

_row_spec = pl.BlockSpec((_BLK, D), lambda i: (i, 0))
_vec_spec = pl.BlockSpec((_BLK, 1), lambda i: (i, 0))
_full_mat = pl.BlockSpec((D, D), lambda i: (0, 0))
_full_vec = pl.BlockSpec((D,), lambda i: (0,))


def _dinv(p0, p1):
    return lax.rsqrt(1.0 + p0 + p1)


def _tc_a_body(x_ref, wl_ref, bl_ref, w1_ref, p0_ref, p1_ref, z_ref):
    t = jnp.dot(x_ref[...], wl_ref[...], preferred_element_type=jnp.float32)
    t = t + bl_ref[...][None, :]
    y = jnp.dot(t, w1_ref[...], preferred_element_type=jnp.float32)
    z_ref[...] = y * _dinv(p0_ref[...], p1_ref[...])


_tc_a = pl.pallas_call(
    _tc_a_body,
    grid=(_GRID,),
    in_specs=[_row_spec, _full_mat, _full_vec, _full_mat, _vec_spec, _vec_spec],
    out_specs=_row_spec,
    out_shape=jax.ShapeDtypeStruct((N_PAD, D), jnp.float32),
)


def _tc_b_body(sa_ref, sb_ref, z1_ref, p0_ref, p1_ref, b1_ref, w2_ref, z2_ref):
    dinv = _dinv(p0_ref[...], p1_ref[...])
    s = sa_ref[...] + sb_ref[...] - z1_ref[...]
    x1 = jnp.maximum(dinv * s + b1_ref[...][None, :], 0.0)
    y2 = jnp.dot(x1, w2_ref[...], preferred_element_type=jnp.float32)
    z2_ref[...] = y2 * dinv


_tc_b = pl.pallas_call(
    _tc_b_body,
    grid=(_GRID,),
    in_specs=[_row_spec, _row_spec, _row_spec, _vec_spec, _vec_spec,
              _full_vec, _full_mat],
    out_specs=_row_spec,
    out_shape=jax.ShapeDtypeStruct((N_PAD, D), jnp.float32),
)


def _tc_c_body(sa_ref, sb_ref, z2_ref, p0_ref, p1_ref, b2_ref, h_ref):
    dinv = _dinv(p0_ref[...], p1_ref[...])
    s = sa_ref[...] + sb_ref[...] - z2_ref[...]
    h_ref[...] = (dinv * s + b2_ref[...][None, :]).astype(jnp.bfloat16)


_tc_c = pl.pallas_call(
    _tc_c_body,
    grid=(_GRID,),
    in_specs=[_row_spec, _row_spec, _row_spec, _vec_spec, _vec_spec, _full_vec],
    out_specs=_row_spec,
    out_shape=jax.ShapeDtypeStruct((N_PAD, D), jnp.bfloat16),
)


def _pack_edges(v, pad_vals):
    return jnp.concatenate([v, pad_vals]).reshape(NW, NCH, K)


def kernel(x_input, edge_index_input, pos_edge_index_input,
           W_lin, b_lin, W1, b1, W2, b2):
    x_pad = jnp.zeros((N_PAD, D), jnp.float32).at[:N].set(x_input)
    pos = pos_edge_index_input.astype(jnp.int32)
    ei = edge_index_input.astype(jnp.int32)
    pad_vals = N + (jnp.arange(E_PAD - E, dtype=jnp.int32) % 16)
    srcs = _pack_edges(pos[0], pad_vals)
    dsts = _pack_edges(pos[1], pad_vals)
    aidx = _pack_edges(ei[0], pad_vals)
    bidx = _pack_edges(ei[1], pad_vals)

    degp = _deg_call(dsts)
    p0, p1 = degp[0][:, None], degp[1][:, None]

    z1 = _tc_a(x_pad, W_lin, b_lin, W1, p0, p1)
    s1 = _scatter_call(z1, srcs, dsts)
    z2 = _tc_b(s1[0], s1[1], z1, p0, p1, b1, W2)
    s2 = _scatter_call(z2, srcs, dsts)
    h2 = _tc_c(s2[0], s2[1], z2, p0, p1, b2)

    logits = _logits_call(h2, aidx.reshape(NW, LNCH, LK),
                          bidx.reshape(NW, LNCH, LK))
    return logits.reshape(-1)[:E]
